# baseline (speedup 1.0000x reference)
import jax
import jax.numpy as jnp
from jax.experimental import pallas as pl

N_POINTS = 8
ENC_DIM = 32
ENC_DEPTH = 4
BLK = 512

_CORNERS = [(0, 0, 0), (1, 0, 0), (0, 1, 0), (0, 0, 1),
            (1, 0, 1), (0, 1, 1), (1, 1, 0), (1, 1, 1)]

_NEMB = ENC_DEPTH * 256
_NME = ENC_DEPTH * 48
_NTAB = _NEMB + _NME


def _rep_lanes(a, r):
    return jnp.repeat(a, r, axis=1)


def _tile_lanes(a, r):
    return jnp.tile(a, (1, r))


def _bbox_block_kernel(idx_ref, inp_ref, table_ref, rep_ref, out_ref):
    j = idx_ref[:, :] + 1
    fbits = jax.lax.bitcast_convert_type(j.astype(jnp.float32), jnp.int32)
    depth = jax.lax.shift_right_logical(fbits, 23) - 127
    depth = depth - (j == (1 << 15)).astype(jnp.int32)

    cols = jax.lax.broadcasted_iota(jnp.int32, (BLK, 4 * 16), 1)
    oh = jnp.zeros((BLK, 4 * 16), jnp.float32)
    for i in range(ENC_DEPTH):
        shift = jnp.maximum(depth - i, 0)
        anc = jnp.where(depth >= i, jax.lax.shift_right_logical(j, shift) - 1, 0)
        oh = oh + (cols == (16 * i + anc)).astype(jnp.float32)
    oh2 = jnp.tile(oh, (1, 2))

    g = jax.lax.dot_general(
        oh2.astype(jnp.bfloat16), table_ref[:, :],
        (((1,), (0,)), ((), ())),
        preferred_element_type=jnp.float32,
    )
    me = g[:, _NEMB:_NEMB + _NME]

    inp = inp_ref[:, :]
    xs = []
    for i in range(ENC_DEPTH):
        nmin24 = me[:, 48 * i:48 * i + 24]
        ext24 = me[:, 48 * i + 24:48 * i + 48]
        xs.append(jnp.clip((inp - nmin24) / ext24, 0.0, 1.0))
    x96 = jnp.concatenate(xs, axis=1)
    x_hi = x96.astype(jnp.bfloat16)
    x_lo = (x96 - x_hi.astype(jnp.float32)).astype(jnp.bfloat16)
    g2 = jax.lax.dot_general(
        jnp.concatenate([x_hi, x_lo], axis=1), rep_ref[:, :],
        (((1,), (0,)), ((), ())),
        preferred_element_type=jnp.float32,
    )

    for i in range(ENC_DEPTH):
        xb = g2[:, 768 * i:768 * i + 256]
        yb = g2[:, 768 * i + 256:768 * i + 512]
        zb = g2[:, 768 * i + 512:768 * i + 768]
        t = [_tile_lanes(g[:, 256 * i + s * ENC_DIM:256 * i + (s + 1) * ENC_DIM],
                         N_POINTS) for s in range(8)]
        u = t[2] + xb * t[3]
        v = t[4] + xb * t[5]
        w = t[6] + xb * t[7]
        t1 = t[0] + xb * t[1]
        t2 = t1 + yb * u
        t3 = v + yb * w
        out_ref[:, i * 256:(i + 1) * 256] = t2 + zb * t3


def _build_table(nodes_min, nodes_extent, emb):
    zeros = jnp.zeros((16, 256), jnp.float32)
    emb16 = emb[:16]
    c = [emb16[:, s * ENC_DIM:(s + 1) * ENC_DIM] for s in range(8)]
    emb16 = jnp.concatenate(
        [c[0],
         c[1] - c[0],
         c[2] - c[0],
         c[6] - c[2] - c[1] + c[0],
         c[3] - c[0],
         c[4] - c[3] - c[1] + c[0],
         c[5] - c[3] - c[2] + c[0],
         c[7] - c[5] - c[4] + c[3] - c[6] + c[2] + c[1] - c[0]],
        axis=1)
    emb_blocks = []
    for i in range(ENC_DEPTH):
        emb_blocks.append(jnp.concatenate(
            [zeros] * i + [emb16] + [zeros] * (ENC_DEPTH - 1 - i), axis=1))
    emb_bd = jnp.concatenate(emb_blocks, axis=0)

    me = jnp.concatenate(
        [jnp.repeat(nodes_min[:16], N_POINTS, axis=1),
         jnp.repeat(nodes_extent[:16], N_POINTS, axis=1)], axis=1)
    mz = jnp.zeros((16, 48), jnp.float32)
    me_blocks = []
    for i in range(ENC_DEPTH):
        me_blocks.append(jnp.concatenate(
            [mz] * i + [me] + [mz] * (ENC_DEPTH - 1 - i), axis=1))
    me_bd = jnp.concatenate(me_blocks, axis=0)

    me_hi = me_bd.astype(jnp.bfloat16)
    me_lo = (me_bd - me_hi.astype(jnp.float32)).astype(jnp.bfloat16)
    emb_hi = emb_bd.astype(jnp.bfloat16)
    emb_lo = (emb_bd - emb_hi.astype(jnp.float32)).astype(jnp.bfloat16)
    hi_rows = jnp.concatenate([emb_hi, me_hi], axis=1)
    lo_rows = jnp.concatenate([emb_lo, me_lo], axis=1)
    return jnp.concatenate([hi_rows, lo_rows], axis=0)


def _build_rep_matrix():
    import numpy as np
    m = np.zeros((192, 3072), np.float32)
    for half in range(2):
        for i in range(ENC_DEPTH):
            for c in range(3):
                for p in range(N_POINTS):
                    r = 96 * half + 24 * i + 8 * c + p
                    base = 768 * i + 256 * c + 32 * p
                    m[r, base:base + 32] = 1.0
    return jnp.asarray(m, jnp.bfloat16)


def kernel(inp, nodes_min, nodes_extent, emb, bbox_idxs):
    n = inp.shape[0]
    inp24 = inp.transpose(0, 2, 1).reshape(n, 3 * N_POINTS)
    idx2 = bbox_idxs.reshape(n, 1)
    table = _build_table(
        nodes_min, nodes_extent, emb)
    repm = _build_rep_matrix()
    grid = (n // BLK,)
    out = pl.pallas_call(
        _bbox_block_kernel,
        grid=grid,
        in_specs=[
            pl.BlockSpec((BLK, 1), lambda i: (i, 0)),
            pl.BlockSpec((BLK, 3 * N_POINTS), lambda i: (i, 0)),
            pl.BlockSpec((128, _NTAB), lambda i: (0, 0)),
            pl.BlockSpec((192, 3072), lambda i: (0, 0)),
        ],
        out_specs=pl.BlockSpec((BLK, 1024), lambda i: (i, 0)),
        out_shape=jax.ShapeDtypeStruct((n, 1024), jnp.float32),
    )(idx2, inp24, table, repm)
    return out


# speedup vs baseline: 1.0048x; 1.0048x over previous
import jax
import jax.numpy as jnp
from jax.experimental import pallas as pl

N_POINTS = 8
ENC_DIM = 32
ENC_DEPTH = 4
BLK = 512

_CORNERS = [(0, 0, 0), (1, 0, 0), (0, 1, 0), (0, 0, 1),
            (1, 0, 1), (0, 1, 1), (1, 1, 0), (1, 1, 1)]

_NEMB = ENC_DEPTH * 256
_NME = ENC_DEPTH * 48
_NTAB = _NEMB + _NME


def _rep_lanes(a, r):
    return jnp.repeat(a, r, axis=1)


def _tile_lanes(a, r):
    return jnp.tile(a, (1, r))


def _split_hi_lo(v):
    bits = jax.lax.bitcast_convert_type(v, jnp.int32)
    hi = jax.lax.bitcast_convert_type(
        jnp.bitwise_and(bits, jnp.int32(-65536)), jnp.float32)
    return hi, (v - hi).astype(jnp.bfloat16)


def _bbox_block_kernel(idx_ref, inp_ref, table_ref, rep_ref, out_ref):
    j = idx_ref[:, :] + 1
    fbits = jax.lax.bitcast_convert_type(j.astype(jnp.float32), jnp.int32)
    depth = jax.lax.shift_right_logical(fbits, 23) - 127
    depth = depth - (j == (1 << 15)).astype(jnp.int32)

    cols = jax.lax.broadcasted_iota(jnp.int32, (BLK, 4 * 16), 1)
    oh = jnp.zeros((BLK, 4 * 16), jnp.float32)
    for i in range(ENC_DEPTH):
        shift = jnp.maximum(depth - i, 0)
        anc = jnp.where(depth >= i, jax.lax.shift_right_logical(j, shift) - 1, 0)
        oh = oh + (cols == (16 * i + anc)).astype(jnp.float32)
    oh2 = jnp.tile(oh, (1, 2))

    g = jax.lax.dot_general(
        oh2.astype(jnp.bfloat16), table_ref[:, :],
        (((1,), (0,)), ((), ())),
        preferred_element_type=jnp.float32,
    )
    me = g[:, _NEMB:_NEMB + _NME]

    inp = inp_ref[:, :]
    xs = []
    for i in range(ENC_DEPTH):
        nmin24 = me[:, 48 * i:48 * i + 24]
        ext24 = me[:, 48 * i + 24:48 * i + 48]
        xs.append(jnp.clip((inp - nmin24) / ext24, 0.0, 1.0))
    x96 = jnp.concatenate(xs, axis=1)
    x_hi_f, x_lo = _split_hi_lo(x96)
    x_hi = x_hi_f.astype(jnp.bfloat16)
    g2 = jax.lax.dot_general(
        jnp.concatenate([x_hi, x_lo], axis=1), rep_ref[:, :],
        (((1,), (0,)), ((), ())),
        preferred_element_type=jnp.float32,
    )

    for i in range(ENC_DEPTH):
        xb = g2[:, 768 * i:768 * i + 256]
        yb = g2[:, 768 * i + 256:768 * i + 512]
        zb = g2[:, 768 * i + 512:768 * i + 768]
        t = [_tile_lanes(g[:, 256 * i + s * ENC_DIM:256 * i + (s + 1) * ENC_DIM],
                         N_POINTS) for s in range(8)]
        u = t[2] + xb * t[3]
        v = t[4] + xb * t[5]
        w = t[6] + xb * t[7]
        t1 = t[0] + xb * t[1]
        t2 = t1 + yb * u
        t3 = v + yb * w
        out_ref[:, i * 256:(i + 1) * 256] = t2 + zb * t3


def _build_table(nodes_min, nodes_extent, emb):
    zeros = jnp.zeros((16, 256), jnp.float32)
    emb16 = emb[:16]
    c = [emb16[:, s * ENC_DIM:(s + 1) * ENC_DIM] for s in range(8)]
    emb16 = jnp.concatenate(
        [c[0],
         c[1] - c[0],
         c[2] - c[0],
         c[6] - c[2] - c[1] + c[0],
         c[3] - c[0],
         c[4] - c[3] - c[1] + c[0],
         c[5] - c[3] - c[2] + c[0],
         c[7] - c[5] - c[4] + c[3] - c[6] + c[2] + c[1] - c[0]],
        axis=1)
    emb_blocks = []
    for i in range(ENC_DEPTH):
        emb_blocks.append(jnp.concatenate(
            [zeros] * i + [emb16] + [zeros] * (ENC_DEPTH - 1 - i), axis=1))
    emb_bd = jnp.concatenate(emb_blocks, axis=0)

    me = jnp.concatenate(
        [jnp.repeat(nodes_min[:16], N_POINTS, axis=1),
         jnp.repeat(nodes_extent[:16], N_POINTS, axis=1)], axis=1)
    mz = jnp.zeros((16, 48), jnp.float32)
    me_blocks = []
    for i in range(ENC_DEPTH):
        me_blocks.append(jnp.concatenate(
            [mz] * i + [me] + [mz] * (ENC_DEPTH - 1 - i), axis=1))
    me_bd = jnp.concatenate(me_blocks, axis=0)

    me_hi_f, me_lo = _split_hi_lo(me_bd)
    emb_hi_f, emb_lo = _split_hi_lo(emb_bd)
    hi_rows = jnp.concatenate(
        [emb_hi_f.astype(jnp.bfloat16), me_hi_f.astype(jnp.bfloat16)], axis=1)
    lo_rows = jnp.concatenate([emb_lo, me_lo], axis=1)
    return jnp.concatenate([hi_rows, lo_rows], axis=0)


def _build_rep_matrix():
    import numpy as np
    m = np.zeros((192, 3072), np.float32)
    for half in range(2):
        for i in range(ENC_DEPTH):
            for c in range(3):
                for p in range(N_POINTS):
                    r = 96 * half + 24 * i + 8 * c + p
                    base = 768 * i + 256 * c + 32 * p
                    m[r, base:base + 32] = 1.0
    return jnp.asarray(m, jnp.bfloat16)


def kernel(inp, nodes_min, nodes_extent, emb, bbox_idxs):
    n = inp.shape[0]
    inp24 = inp.transpose(0, 2, 1).reshape(n, 3 * N_POINTS)
    idx2 = bbox_idxs.reshape(n, 1)
    table = _build_table(
        nodes_min, nodes_extent, emb)
    repm = _build_rep_matrix()
    grid = (n // BLK,)
    out = pl.pallas_call(
        _bbox_block_kernel,
        grid=grid,
        in_specs=[
            pl.BlockSpec((BLK, 1), lambda i: (i, 0)),
            pl.BlockSpec((BLK, 3 * N_POINTS), lambda i: (i, 0)),
            pl.BlockSpec((128, _NTAB), lambda i: (0, 0)),
            pl.BlockSpec((192, 3072), lambda i: (0, 0)),
        ],
        out_specs=pl.BlockSpec((BLK, 1024), lambda i: (i, 0)),
        out_shape=jax.ShapeDtypeStruct((n, 1024), jnp.float32),
    )(idx2, inp24, table, repm)
    return out
